# Initial kernel scaffold; baseline (speedup 1.0000x reference)
import jax
import jax.numpy as jnp
from jax.experimental import pallas as pl


def kernel(x, edge_index, batch, Wl1, bl1, Wr1, w1, Wl2, bl2, Wr2, w2, Wl3, bl3, Wr3, w3, W1, lb1, W2, lb2, W3, lb3):
    raise NotImplementedError("write your pallas kernel here")


# speedup vs baseline: 7.7180x; 7.7180x over previous
import functools

import jax
import jax.numpy as jnp
from jax import lax
from jax.experimental import pallas as pl
from jax.experimental.pallas import tpu as pltpu
from jax.experimental.pallas import tpu_sc as plsc

NG = 64
RATIO = 0.8
NC = 2
NS = 16
NW = NC * NS
CHUNK = 128


def _make_aggregate(NP, F, CPT):
  RPT = NP // NS

  mesh = plsc.VectorSubcoreMesh(core_axis_name="c", subcore_axis_name="s")

  @functools.partial(
      pl.kernel,
      out_type=(
          jax.ShapeDtypeStruct((NC, NP, F), jnp.float32),
          jax.ShapeDtypeStruct((NC, NP), jnp.float32),
      ),
      mesh=mesh,
      scratch_types=[
          pltpu.VMEM((CPT, CHUNK), jnp.int32),
          pltpu.VMEM((CPT, CHUNK), jnp.int32),
          pltpu.VMEM((CHUNK, F), jnp.float32),
          pltpu.VMEM((CHUNK,), jnp.float32),
          pltpu.VMEM_SHARED((NP, F), jnp.float32),
          pltpu.VMEM_SHARED((NP,), jnp.float32),
          pltpu.SemaphoreType.DMA,
          pltpu.SemaphoreType.DMA,
      ],
  )
  def agg_kernel(x_hbm, f_hbm, srcg, dstg, zf_hbm, z1_hbm,
                 aggp, degp,
                 src_v, dst_v, rows_v, fv, agg_sh, deg_sh, sem, sem2):
    c = lax.axis_index("c")
    s = lax.axis_index("s")
    wid = c * NS + s

    pltpu.sync_copy(zf_hbm.at[pl.ds(s * RPT, RPT), :],
                    agg_sh.at[pl.ds(s * RPT, RPT), :])
    pltpu.sync_copy(z1_hbm.at[pl.ds(s * RPT, RPT)],
                    deg_sh.at[pl.ds(s * RPT, RPT)])
    pltpu.sync_copy(srcg.at[wid], src_v)
    pltpu.sync_copy(dstg.at[wid], dst_v)
    plsc.subcore_barrier()

    @pl.loop(0, CPT)
    def _(i):
      pltpu.async_copy(x_hbm.at[src_v.at[i]], rows_v, sem).wait()
      pltpu.async_copy(f_hbm.at[src_v.at[i]], fv, sem2).wait()
      pltpu.sync_copy(rows_v, agg_sh.at[dst_v.at[i]], add=True)
      pltpu.sync_copy(fv, deg_sh.at[dst_v.at[i]], add=True)

    plsc.subcore_barrier()
    pltpu.sync_copy(agg_sh.at[pl.ds(s * RPT, RPT), :],
                    aggp.at[c, pl.ds(s * RPT, RPT), :])
    pltpu.sync_copy(deg_sh.at[pl.ds(s * RPT, RPT)],
                    degp.at[c, pl.ds(s * RPT, RPT)])

  return agg_kernel


def _linear_block(agg0, agg1, deg0, deg1, x, wlt, bl, wrt, wsc, h_out, s_out):
  deg = jnp.maximum(deg0[...] + deg1[...], 1.0)
  agg = (agg0[...] + agg1[...]) / deg
  h = jnp.dot(agg, wlt[...], preferred_element_type=jnp.float32)
  h = h + bl[...] + jnp.dot(x[...], wrt[...], preferred_element_type=jnp.float32)
  h = jnp.maximum(h, 0.0)
  h_out[...] = h
  s_out[...] = jnp.tanh(jnp.dot(h, wsc[...], preferred_element_type=jnp.float32))


def _linear(aggp, degp, x, wlt, bl, wrt, wsc, NP, F, BN=512):
  grid = (NP // BN,)
  return pl.pallas_call(
      _linear_block,
      grid=grid,
      in_specs=[
          pl.BlockSpec((None, BN, F), lambda i: (0, i, 0)),
          pl.BlockSpec((None, BN, F), lambda i: (1, i, 0)),
          pl.BlockSpec((None, BN, 1), lambda i: (0, i, 0)),
          pl.BlockSpec((None, BN, 1), lambda i: (1, i, 0)),
          pl.BlockSpec((BN, F), lambda i: (i, 0)),
          pl.BlockSpec((F, F), lambda i: (0, 0)),
          pl.BlockSpec((1, F), lambda i: (0, 0)),
          pl.BlockSpec((F, F), lambda i: (0, 0)),
          pl.BlockSpec((F, 1), lambda i: (0, 0)),
      ],
      out_specs=[
          pl.BlockSpec((BN, F), lambda i: (i, 0)),
          pl.BlockSpec((BN, 1), lambda i: (i, 0)),
      ],
      out_shape=[
          jax.ShapeDtypeStruct((NP, F), jnp.float32),
          jax.ShapeDtypeStruct((NP, 1), jnp.float32),
      ],
  )(aggp, aggp, degp, degp, x, wlt, bl, wrt, wsc)


def _rank_block(srow, brow, vrow, scol, bcol, vcol, keep_out, rank_s, n_s,
                *, BR, BJ, NJ):
  i = pl.program_id(0)
  j = pl.program_id(1)

  @pl.when(j == 0)
  def _():
    rank_s[...] = jnp.zeros_like(rank_s)
    n_s[...] = jnp.zeros_like(n_s)

  si = srow[...]
  bi = brow[...]
  sj = scol[...]
  bj = bcol[...]
  vj = vcol[...] > 0.0
  ridx = i * BR + lax.broadcasted_iota(jnp.int32, (BR, 1), 0)
  cidx = j * BJ + lax.broadcasted_iota(jnp.int32, (1, BJ), 1)
  same = (bj == bi) & vj
  ahead = (sj > si) | ((sj == si) & (cidx < ridx))
  rank_s[...] += jnp.sum((same & ahead).astype(jnp.float32), axis=1,
                         keepdims=True)
  n_s[...] += jnp.sum(same.astype(jnp.float32), axis=1, keepdims=True)

  @pl.when(j == NJ - 1)
  def _():
    k = jnp.ceil(jnp.float32(RATIO) * n_s[...])
    keep = (vrow[...] > 0.0) & (rank_s[...] < k)
    keep_out[...] = keep.astype(jnp.float32)


def _rank(score, batch, valid, NP, BR=512, BJ=1024):
  NI, NJ = NP // BR, NP // BJ
  scol = score.reshape(1, NP)
  bcol = batch.reshape(1, NP)
  vcol = valid.reshape(1, NP)
  return pl.pallas_call(
      functools.partial(_rank_block, BR=BR, BJ=BJ, NJ=NJ),
      grid=(NI, NJ),
      in_specs=[
          pl.BlockSpec((BR, 1), lambda i, j: (i, 0)),
          pl.BlockSpec((BR, 1), lambda i, j: (i, 0)),
          pl.BlockSpec((BR, 1), lambda i, j: (i, 0)),
          pl.BlockSpec((1, BJ), lambda i, j: (0, j)),
          pl.BlockSpec((1, BJ), lambda i, j: (0, j)),
          pl.BlockSpec((1, BJ), lambda i, j: (0, j)),
      ],
      out_specs=pl.BlockSpec((BR, 1), lambda i, j: (i, 0)),
      out_shape=jax.ShapeDtypeStruct((NP, 1), jnp.float32),
      scratch_shapes=[
          pltpu.VMEM((BR, 1), jnp.float32),
          pltpu.VMEM((BR, 1), jnp.float32),
      ],
  )(score, batch.reshape(NP, 1), valid, scol, bcol, vcol)


def _readout_block(h, keep, score, batch, xn_out, ro_out, mx_s, sm_s, cnt_s,
                   *, NB, F):
  i = pl.program_id(0)

  @pl.when(i == 0)
  def _():
    mx_s[...] = jnp.full_like(mx_s, -jnp.inf)
    sm_s[...] = jnp.zeros_like(sm_s)
    cnt_s[...] = jnp.zeros_like(cnt_s)

  kp = keep[...] > 0.0
  xn = jnp.where(kp, h[...] * score[...], 0.0)
  xn_out[...] = xn
  b = batch[...]
  gids = lax.broadcasted_iota(jnp.int32, (1, NG), 1)
  onehot = ((b == gids) & kp).astype(jnp.float32)
  sm_s[...] += lax.dot_general(onehot, xn, (((0,), (0,)), ((), ())),
                               preferred_element_type=jnp.float32)
  cnt_s[...] += lax.dot_general(onehot, jnp.ones_like(keep[...]),
                                (((0,), (0,)), ((), ())),
                                preferred_element_type=jnp.float32)
  for g in range(NG):
    mask = (b == g) & kp
    mval = jnp.max(jnp.where(mask, xn, -jnp.inf), axis=0, keepdims=True)
    mx_s[g:g + 1, :] = jnp.maximum(mx_s[g:g + 1, :], mval)

  @pl.when(i == NB - 1)
  def _():
    mean = sm_s[...] / jnp.maximum(cnt_s[...], 1.0)
    ro_out[:, :F] = mx_s[...]
    ro_out[:, F:] = mean


def _readout(h, keep, score, batch, NP, F, BD=512):
  NB = NP // BD
  return pl.pallas_call(
      functools.partial(_readout_block, NB=NB, F=F),
      grid=(NB,),
      in_specs=[
          pl.BlockSpec((BD, F), lambda i: (i, 0)),
          pl.BlockSpec((BD, 1), lambda i: (i, 0)),
          pl.BlockSpec((BD, 1), lambda i: (i, 0)),
          pl.BlockSpec((BD, 1), lambda i: (i, 0)),
      ],
      out_specs=[
          pl.BlockSpec((BD, F), lambda i: (i, 0)),
          pl.BlockSpec((NG, 2 * F), lambda i: (0, 0)),
      ],
      out_shape=[
          jax.ShapeDtypeStruct((NP, F), jnp.float32),
          jax.ShapeDtypeStruct((NG, 2 * F), jnp.float32),
      ],
      scratch_shapes=[
          pltpu.VMEM((NG, F), jnp.float32),
          pltpu.VMEM((NG, F), jnp.float32),
          pltpu.VMEM((NG, 1), jnp.float32),
      ],
  )(h, keep, score, batch)


def _head_block(x1, x2, x3, w1t, b1, w2t, b2, w3t, b3, out):
  z = x1[...] + x2[...] + x3[...]
  z = jnp.maximum(jnp.dot(z, w1t[...], preferred_element_type=jnp.float32)
                  + b1[...], 0.0)
  z = jnp.maximum(jnp.dot(z, w2t[...], preferred_element_type=jnp.float32)
                  + b2[...], 0.0)
  z = jnp.maximum(jnp.dot(z, w3t[...], preferred_element_type=jnp.float32)
                  + b3[...], 0.0)
  out[...] = z


def _head(x1, x2, x3, W1, lb1, W2, lb2, W3, lb3):
  OUT = W3.shape[0]
  return pl.pallas_call(
      _head_block,
      out_shape=jax.ShapeDtypeStruct((NG, OUT), jnp.float32),
  )(x1, x2, x3, W1.T, lb1.reshape(1, -1), W2.T, lb2.reshape(1, -1),
    W3.T, lb3.reshape(1, -1))


def kernel(x, edge_index, batch, Wl1, bl1, Wr1, w1, Wl2, bl2, Wr2, w2,
           Wl3, bl3, Wr3, w3, W1, lb1, W2, lb2, W3, lb3):
  N = x.shape[0]
  F = x.shape[-1]
  E = edge_index.shape[1]

  NP = ((N + 1 + 511) // 512) * 512
  CPT = (E + NW * CHUNK - 1) // (NW * CHUNK)
  EP = NW * CPT * CHUNK

  xp = jnp.zeros((NP, F), jnp.float32).at[:N].set(x.reshape(N, F))
  batch_p = jnp.zeros((NP,), jnp.int32).at[:N].set(batch.astype(jnp.int32))
  valid = (jnp.arange(NP) < N).astype(jnp.float32)
  padidx = (N + (jnp.arange(EP - E) % min(CHUNK, NP - N - 1))).astype(jnp.int32)
  src = jnp.concatenate([edge_index[0].astype(jnp.int32), padidx])
  dst = jnp.concatenate([edge_index[1].astype(jnp.int32), padidx])
  srcg = src.reshape(NW, CPT, CHUNK)
  dstg = dst.reshape(NW, CPT, CHUNK)
  zf = jnp.zeros((NP, F), jnp.float32)
  z1 = jnp.zeros((NP,), jnp.float32)

  aggregate = _make_aggregate(NP, F, CPT)

  def layer(xc, fc, Wl, bl, Wr, w):
    aggp, degp = aggregate(xc, fc, srcg, dstg, zf, z1)
    wsc = (w / jnp.linalg.norm(w)).reshape(F, 1)
    h, score = _linear(aggp, degp.reshape(NC, NP, 1), xc, Wl.T,
                       bl.reshape(1, F), Wr.T, wsc, NP, F)
    keep = _rank(score, batch_p, fc.reshape(NP, 1), NP)
    xn, ro = _readout(h, keep, score, batch_p.reshape(NP, 1), NP, F)
    return xn, keep.reshape(NP), ro

  h1, v1, x1 = layer(xp, valid, Wl1, bl1, Wr1, w1)
  h2, v2, x2 = layer(h1, v1, Wl2, bl2, Wr2, w2)
  _, _, x3 = layer(h2, v2, Wl3, bl3, Wr3, w3)
  return _head(x1, x2, x3, W1, lb1, W2, lb2, W3, lb3)


# speedup vs baseline: 14.0692x; 1.8229x over previous
import functools

import jax
import jax.numpy as jnp
from jax import lax
from jax.experimental import pallas as pl
from jax.experimental.pallas import tpu as pltpu
from jax.experimental.pallas import tpu_sc as plsc

NG = 64
RATIO = 0.8
NC = 2
NS = 16
NW = NC * NS
CHUNK = 128


def _make_aggregate(NP, F, CPT):
  RPT = NP // NS

  mesh = plsc.VectorSubcoreMesh(core_axis_name="c", subcore_axis_name="s")

  @functools.partial(
      pl.kernel,
      out_type=(
          jax.ShapeDtypeStruct((NC, NP, F), jnp.float32),
          jax.ShapeDtypeStruct((NC, NP), jnp.float32),
      ),
      mesh=mesh,
      scratch_types=[
          pltpu.VMEM((2, CHUNK), jnp.int32),
          pltpu.VMEM((2, CHUNK), jnp.int32),
          pltpu.VMEM((2, CHUNK, F), jnp.float32),
          pltpu.VMEM((2, CHUNK), jnp.float32),
          pltpu.VMEM_SHARED((NP, F), jnp.float32),
          pltpu.VMEM_SHARED((NP,), jnp.float32),
          pltpu.SemaphoreType.DMA((2,)),
          pltpu.SemaphoreType.DMA((2,)),
          pltpu.SemaphoreType.DMA((2,)),
      ],
  )
  def agg_kernel(x_hbm, f_hbm, srcg, dstg, zf_hbm, z1_hbm,
                 aggp, degp,
                 src_v, dst_v, rows_v, fv, agg_sh, deg_sh, sem, sem2, sem3):
    c = lax.axis_index("c")
    s = lax.axis_index("s")
    wid = c * NS + s

    pltpu.sync_copy(zf_hbm.at[pl.ds(s * RPT, RPT), :],
                    agg_sh.at[pl.ds(s * RPT, RPT), :])
    pltpu.sync_copy(z1_hbm.at[pl.ds(s * RPT, RPT)],
                    deg_sh.at[pl.ds(s * RPT, RPT)])
    plsc.subcore_barrier()

    def idx_issue(i, b):
      pltpu.async_copy(srcg.at[wid, i], src_v.at[b], sem3.at[b])
      pltpu.async_copy(dstg.at[wid, i], dst_v.at[b], sem3.at[b])

    def idx_wait(i, b):
      pltpu.make_async_copy(srcg.at[wid, i], src_v.at[b], sem3.at[b]).wait()
      pltpu.make_async_copy(dstg.at[wid, i], dst_v.at[b], sem3.at[b]).wait()

    def g_issue(i, b):
      pltpu.async_copy(x_hbm.at[src_v.at[b]], rows_v.at[b], sem.at[b])
      pltpu.async_copy(f_hbm.at[src_v.at[b]], fv.at[b], sem2.at[b])

    def g_wait(i, b):
      pltpu.make_async_copy(x_hbm.at[src_v.at[b]], rows_v.at[b],
                            sem.at[b]).wait()
      pltpu.make_async_copy(f_hbm.at[src_v.at[b]], fv.at[b],
                            sem2.at[b]).wait()

    idx_issue(0, 0)
    idx_wait(0, 0)
    g_issue(0, 0)

    @pl.loop(0, CPT // 2)
    def _(ih):
      for b in range(2):
        i = ih * 2 + b

        @pl.when(i < CPT - 1)
        def _():
          idx_issue(i + 1, 1 - b)

        g_wait(i, b)

        @pl.when(i < CPT - 1)
        def _():
          idx_wait(i + 1, 1 - b)
          g_issue(i + 1, 1 - b)

        pltpu.sync_copy(rows_v.at[b], agg_sh.at[dst_v.at[b]], add=True)
        pltpu.sync_copy(fv.at[b], deg_sh.at[dst_v.at[b]], add=True)

    plsc.subcore_barrier()
    pltpu.sync_copy(agg_sh.at[pl.ds(s * RPT, RPT), :],
                    aggp.at[c, pl.ds(s * RPT, RPT), :])
    pltpu.sync_copy(deg_sh.at[pl.ds(s * RPT, RPT)],
                    degp.at[c, pl.ds(s * RPT, RPT)])

  return agg_kernel


def _linear_block(agg0, agg1, deg0, deg1, x, wlt, bl, wrt, wsc, h_out, s_out):
  deg = jnp.maximum(deg0[...] + deg1[...], 1.0)
  agg = (agg0[...] + agg1[...]) / deg
  h = jnp.dot(agg, wlt[...], preferred_element_type=jnp.float32)
  h = h + bl[...] + jnp.dot(x[...], wrt[...], preferred_element_type=jnp.float32)
  h = jnp.maximum(h, 0.0)
  h_out[...] = h
  s_out[...] = jnp.tanh(jnp.dot(h, wsc[...], preferred_element_type=jnp.float32))


def _linear(aggp, degp, x, wlt, bl, wrt, wsc, NP, F, BN=512):
  grid = (NP // BN,)
  return pl.pallas_call(
      _linear_block,
      grid=grid,
      in_specs=[
          pl.BlockSpec((None, BN, F), lambda i: (0, i, 0)),
          pl.BlockSpec((None, BN, F), lambda i: (1, i, 0)),
          pl.BlockSpec((None, BN, 1), lambda i: (0, i, 0)),
          pl.BlockSpec((None, BN, 1), lambda i: (1, i, 0)),
          pl.BlockSpec((BN, F), lambda i: (i, 0)),
          pl.BlockSpec((F, F), lambda i: (0, 0)),
          pl.BlockSpec((1, F), lambda i: (0, 0)),
          pl.BlockSpec((F, F), lambda i: (0, 0)),
          pl.BlockSpec((F, 1), lambda i: (0, 0)),
      ],
      out_specs=[
          pl.BlockSpec((BN, F), lambda i: (i, 0)),
          pl.BlockSpec((BN, 1), lambda i: (i, 0)),
      ],
      out_shape=[
          jax.ShapeDtypeStruct((NP, F), jnp.float32),
          jax.ShapeDtypeStruct((NP, 1), jnp.float32),
      ],
  )(aggp, aggp, degp, degp, x, wlt, bl, wrt, wsc)


def _rank_block(srow, brow, vrow, scol, bcol, vcol, keep_out, rank_s, n_s,
                *, BR, BJ, NJ):
  i = pl.program_id(0)
  j = pl.program_id(1)

  @pl.when(j == 0)
  def _():
    rank_s[...] = jnp.zeros_like(rank_s)
    n_s[...] = jnp.zeros_like(n_s)

  bi = brow[...]
  bj = bcol[...]

  @pl.when((jnp.min(bj) <= jnp.max(bi)) & (jnp.max(bj) >= jnp.min(bi)))
  def _():
    si = srow[...]
    sj = scol[...]
    vj = vcol[...] > 0.0
    ridx = i * BR + lax.broadcasted_iota(jnp.int32, (BR, 1), 0)
    cidx = j * BJ + lax.broadcasted_iota(jnp.int32, (1, BJ), 1)
    same = (bj == bi) & vj
    ahead = (sj > si) | ((sj == si) & (cidx < ridx))
    rank_s[...] += jnp.sum((same & ahead).astype(jnp.float32), axis=1,
                           keepdims=True)
    n_s[...] += jnp.sum(same.astype(jnp.float32), axis=1, keepdims=True)

  @pl.when(j == NJ - 1)
  def _():
    k = jnp.ceil(jnp.float32(RATIO) * n_s[...])
    keep = (vrow[...] > 0.0) & (rank_s[...] < k)
    keep_out[...] = keep.astype(jnp.float32)


def _rank(score, batch, valid, NP, BR=512, BJ=1024):
  NI, NJ = NP // BR, NP // BJ
  scol = score.reshape(1, NP)
  bcol = batch.reshape(1, NP)
  vcol = valid.reshape(1, NP)
  return pl.pallas_call(
      functools.partial(_rank_block, BR=BR, BJ=BJ, NJ=NJ),
      grid=(NI, NJ),
      in_specs=[
          pl.BlockSpec((BR, 1), lambda i, j: (i, 0)),
          pl.BlockSpec((BR, 1), lambda i, j: (i, 0)),
          pl.BlockSpec((BR, 1), lambda i, j: (i, 0)),
          pl.BlockSpec((1, BJ), lambda i, j: (0, j)),
          pl.BlockSpec((1, BJ), lambda i, j: (0, j)),
          pl.BlockSpec((1, BJ), lambda i, j: (0, j)),
      ],
      out_specs=pl.BlockSpec((BR, 1), lambda i, j: (i, 0)),
      out_shape=jax.ShapeDtypeStruct((NP, 1), jnp.float32),
      scratch_shapes=[
          pltpu.VMEM((BR, 1), jnp.float32),
          pltpu.VMEM((BR, 1), jnp.float32),
      ],
  )(score, batch.reshape(NP, 1), valid, scol, bcol, vcol)


def _readout_block(h, keep, score, batch, xn_out, ro_out, mx_s, sm_s, cnt_s,
                   *, NB, F):
  i = pl.program_id(0)

  @pl.when(i == 0)
  def _():
    mx_s[...] = jnp.full_like(mx_s, -jnp.inf)
    sm_s[...] = jnp.zeros_like(sm_s)
    cnt_s[...] = jnp.zeros_like(cnt_s)

  kp = keep[...] > 0.0
  xn = jnp.where(kp, h[...] * score[...], 0.0)
  xn_out[...] = xn
  b = batch[...]
  gids = lax.broadcasted_iota(jnp.int32, (1, NG), 1)
  onehot = ((b == gids) & kp).astype(jnp.float32)
  sm_s[...] += lax.dot_general(onehot, xn, (((0,), (0,)), ((), ())),
                               preferred_element_type=jnp.float32)
  cnt_s[...] += lax.dot_general(onehot, jnp.ones_like(keep[...]),
                                (((0,), (0,)), ((), ())),
                                preferred_element_type=jnp.float32)
  blo = jnp.min(b)
  bhi = jnp.max(b)
  for g in range(NG):
    @pl.when((g >= blo) & (g <= bhi))
    def _(g=g):
      mask = (b == g) & kp
      mval = jnp.max(jnp.where(mask, xn, -jnp.inf), axis=0, keepdims=True)
      mx_s[g:g + 1, :] = jnp.maximum(mx_s[g:g + 1, :], mval)

  @pl.when(i == NB - 1)
  def _():
    mean = sm_s[...] / jnp.maximum(cnt_s[...], 1.0)
    ro_out[:, :F] = mx_s[...]
    ro_out[:, F:] = mean


def _readout(h, keep, score, batch, NP, F, BD=512):
  NB = NP // BD
  return pl.pallas_call(
      functools.partial(_readout_block, NB=NB, F=F),
      grid=(NB,),
      in_specs=[
          pl.BlockSpec((BD, F), lambda i: (i, 0)),
          pl.BlockSpec((BD, 1), lambda i: (i, 0)),
          pl.BlockSpec((BD, 1), lambda i: (i, 0)),
          pl.BlockSpec((BD, 1), lambda i: (i, 0)),
      ],
      out_specs=[
          pl.BlockSpec((BD, F), lambda i: (i, 0)),
          pl.BlockSpec((NG, 2 * F), lambda i: (0, 0)),
      ],
      out_shape=[
          jax.ShapeDtypeStruct((NP, F), jnp.float32),
          jax.ShapeDtypeStruct((NG, 2 * F), jnp.float32),
      ],
      scratch_shapes=[
          pltpu.VMEM((NG, F), jnp.float32),
          pltpu.VMEM((NG, F), jnp.float32),
          pltpu.VMEM((NG, 1), jnp.float32),
      ],
  )(h, keep, score, batch)


def _head_block(x1, x2, x3, w1t, b1, w2t, b2, w3t, b3, out):
  z = x1[...] + x2[...] + x3[...]
  z = jnp.maximum(jnp.dot(z, w1t[...], preferred_element_type=jnp.float32)
                  + b1[...], 0.0)
  z = jnp.maximum(jnp.dot(z, w2t[...], preferred_element_type=jnp.float32)
                  + b2[...], 0.0)
  z = jnp.maximum(jnp.dot(z, w3t[...], preferred_element_type=jnp.float32)
                  + b3[...], 0.0)
  out[...] = z


def _head(x1, x2, x3, W1, lb1, W2, lb2, W3, lb3):
  OUT = W3.shape[0]
  return pl.pallas_call(
      _head_block,
      out_shape=jax.ShapeDtypeStruct((NG, OUT), jnp.float32),
  )(x1, x2, x3, W1.T, lb1.reshape(1, -1), W2.T, lb2.reshape(1, -1),
    W3.T, lb3.reshape(1, -1))


def kernel(x, edge_index, batch, Wl1, bl1, Wr1, w1, Wl2, bl2, Wr2, w2,
           Wl3, bl3, Wr3, w3, W1, lb1, W2, lb2, W3, lb3):
  N = x.shape[0]
  F = x.shape[-1]
  E = edge_index.shape[1]

  NP = ((N + 1 + 511) // 512) * 512
  CPT = (E + NW * CHUNK - 1) // (NW * CHUNK)
  CPT = CPT + (CPT % 2)
  EP = NW * CPT * CHUNK

  xp = jnp.zeros((NP, F), jnp.float32).at[:N].set(x.reshape(N, F))
  batch_p = jnp.full((NP,), NG - 1, jnp.int32).at[:N].set(
      batch.astype(jnp.int32))
  valid = (jnp.arange(NP) < N).astype(jnp.float32)
  padidx = (N + (jnp.arange(EP - E) % min(CHUNK, NP - N - 1))).astype(jnp.int32)
  src = jnp.concatenate([edge_index[0].astype(jnp.int32), padidx])
  dst = jnp.concatenate([edge_index[1].astype(jnp.int32), padidx])
  srcg = src.reshape(NW, CPT, CHUNK)
  dstg = dst.reshape(NW, CPT, CHUNK)
  zf = jnp.zeros((NP, F), jnp.float32)
  z1 = jnp.zeros((NP,), jnp.float32)

  aggregate = _make_aggregate(NP, F, CPT)

  def layer(xc, fc, Wl, bl, Wr, w):
    aggp, degp = aggregate(xc, fc, srcg, dstg, zf, z1)
    wsc = (w / jnp.linalg.norm(w)).reshape(F, 1)
    h, score = _linear(aggp, degp.reshape(NC, NP, 1), xc, Wl.T,
                       bl.reshape(1, F), Wr.T, wsc, NP, F)
    keep = _rank(score, batch_p, fc.reshape(NP, 1), NP)
    xn, ro = _readout(h, keep, score, batch_p.reshape(NP, 1), NP, F)
    return xn, keep.reshape(NP), ro

  h1, v1, x1 = layer(xp, valid, Wl1, bl1, Wr1, w1)
  h2, v2, x2 = layer(h1, v1, Wl2, bl2, Wr2, w2)
  _, _, x3 = layer(h2, v2, Wl3, bl3, Wr3, w3)
  return _head(x1, x2, x3, W1, lb1, W2, lb2, W3, lb3)


# speedup vs baseline: 15.5943x; 1.1084x over previous
import functools

import jax
import jax.numpy as jnp
from jax import lax
from jax.experimental import pallas as pl
from jax.experimental.pallas import tpu as pltpu
from jax.experimental.pallas import tpu_sc as plsc

NG = 64
RATIO = 0.8
NC = 2
NS = 16
NW = NC * NS
CHUNK = 128


def _make_aggregate(NP, F, CPT):
  RPT = NP // NS

  mesh = plsc.VectorSubcoreMesh(core_axis_name="c", subcore_axis_name="s")

  @functools.partial(
      pl.kernel,
      out_type=(
          jax.ShapeDtypeStruct((NC, NP, F), jnp.float32),
          jax.ShapeDtypeStruct((NC, NP), jnp.float32),
      ),
      mesh=mesh,
      scratch_types=[
          pltpu.VMEM((2, CHUNK), jnp.int32),
          pltpu.VMEM((2, CHUNK), jnp.int32),
          pltpu.VMEM((2, CHUNK, F), jnp.float32),
          pltpu.VMEM((2, CHUNK), jnp.float32),
          pltpu.VMEM_SHARED((NP, F), jnp.float32),
          pltpu.VMEM_SHARED((NP,), jnp.float32),
          pltpu.SemaphoreType.DMA((2,)),
          pltpu.SemaphoreType.DMA((2,)),
          pltpu.SemaphoreType.DMA((2,)),
      ],
  )
  def agg_kernel(x_hbm, f_hbm, srcg, dstg, zf_hbm, z1_hbm,
                 aggp, degp,
                 src_v, dst_v, rows_v, fv, agg_sh, deg_sh, sem, sem2, sem3):
    c = lax.axis_index("c")
    s = lax.axis_index("s")
    wid = c * NS + s

    pltpu.sync_copy(zf_hbm.at[pl.ds(s * RPT, RPT), :],
                    agg_sh.at[pl.ds(s * RPT, RPT), :])
    pltpu.sync_copy(z1_hbm.at[pl.ds(s * RPT, RPT)],
                    deg_sh.at[pl.ds(s * RPT, RPT)])
    plsc.subcore_barrier()

    def idx_issue(i, b):
      pltpu.async_copy(srcg.at[wid, i], src_v.at[b], sem3.at[b])
      pltpu.async_copy(dstg.at[wid, i], dst_v.at[b], sem3.at[b])

    def idx_wait(i, b):
      pltpu.make_async_copy(srcg.at[wid, i], src_v.at[b], sem3.at[b]).wait()
      pltpu.make_async_copy(dstg.at[wid, i], dst_v.at[b], sem3.at[b]).wait()

    def g_issue(i, b):
      pltpu.async_copy(x_hbm.at[src_v.at[b]], rows_v.at[b], sem.at[b])
      pltpu.async_copy(f_hbm.at[src_v.at[b]], fv.at[b], sem2.at[b])

    def g_wait(i, b):
      pltpu.make_async_copy(x_hbm.at[src_v.at[b]], rows_v.at[b],
                            sem.at[b]).wait()
      pltpu.make_async_copy(f_hbm.at[src_v.at[b]], fv.at[b],
                            sem2.at[b]).wait()

    idx_issue(0, 0)
    idx_wait(0, 0)
    g_issue(0, 0)

    @pl.loop(0, CPT // 2)
    def _(ih):
      for b in range(2):
        i = ih * 2 + b

        @pl.when(i < CPT - 1)
        def _():
          idx_issue(i + 1, 1 - b)

        g_wait(i, b)

        @pl.when(i < CPT - 1)
        def _():
          idx_wait(i + 1, 1 - b)
          g_issue(i + 1, 1 - b)

        pltpu.sync_copy(rows_v.at[b], agg_sh.at[dst_v.at[b]], add=True)
        pltpu.sync_copy(fv.at[b], deg_sh.at[dst_v.at[b]], add=True)

    plsc.subcore_barrier()
    pltpu.sync_copy(agg_sh.at[pl.ds(s * RPT, RPT), :],
                    aggp.at[c, pl.ds(s * RPT, RPT), :])
    pltpu.sync_copy(deg_sh.at[pl.ds(s * RPT, RPT)],
                    degp.at[c, pl.ds(s * RPT, RPT)])

  return agg_kernel


def _linear_block(agg0, agg1, deg0, deg1, x, wlt, bl, wrt, wsc, h_out, s_out):
  deg = jnp.maximum(deg0[...] + deg1[...], 1.0)
  agg = (agg0[...] + agg1[...]) / deg
  h = jnp.dot(agg, wlt[...], preferred_element_type=jnp.float32)
  h = h + bl[...] + jnp.dot(x[...], wrt[...], preferred_element_type=jnp.float32)
  h = jnp.maximum(h, 0.0)
  h_out[...] = h
  s_out[...] = jnp.tanh(jnp.dot(h, wsc[...], preferred_element_type=jnp.float32))


def _linear(aggp, degp, x, wlt, bl, wrt, wsc, NP, F, BN=512):
  grid = (NP // BN,)
  return pl.pallas_call(
      _linear_block,
      grid=grid,
      in_specs=[
          pl.BlockSpec((None, BN, F), lambda i: (0, i, 0)),
          pl.BlockSpec((None, BN, F), lambda i: (1, i, 0)),
          pl.BlockSpec((None, BN, 1), lambda i: (0, i, 0)),
          pl.BlockSpec((None, BN, 1), lambda i: (1, i, 0)),
          pl.BlockSpec((BN, F), lambda i: (i, 0)),
          pl.BlockSpec((F, F), lambda i: (0, 0)),
          pl.BlockSpec((1, F), lambda i: (0, 0)),
          pl.BlockSpec((F, F), lambda i: (0, 0)),
          pl.BlockSpec((F, 1), lambda i: (0, 0)),
      ],
      out_specs=[
          pl.BlockSpec((BN, F), lambda i: (i, 0)),
          pl.BlockSpec((BN, 1), lambda i: (i, 0)),
      ],
      out_shape=[
          jax.ShapeDtypeStruct((NP, F), jnp.float32),
          jax.ShapeDtypeStruct((NP, 1), jnp.float32),
      ],
  )(aggp, aggp, degp, degp, x, wlt, bl, wrt, wsc)


def _rank_block(srow, brow, vrow, scol, bcol, vcol, keep_out, rank_s, n_s,
                *, BR, BJ, NJ):
  i = pl.program_id(0)
  j = pl.program_id(1)

  @pl.when(j == 0)
  def _():
    rank_s[...] = jnp.zeros_like(rank_s)
    n_s[...] = jnp.zeros_like(n_s)

  bi = brow[...]
  bj = bcol[...]

  @pl.when((jnp.min(bj) <= jnp.max(bi)) & (jnp.max(bj) >= jnp.min(bi)))
  def _():
    si = srow[...]
    sj = scol[...]
    vj = vcol[...] > 0.0
    ridx = i * BR + lax.broadcasted_iota(jnp.int32, (BR, 1), 0)
    cidx = j * BJ + lax.broadcasted_iota(jnp.int32, (1, BJ), 1)
    same = (bj == bi) & vj
    ahead = (sj > si) | ((sj == si) & (cidx < ridx))
    rank_s[...] += jnp.sum((same & ahead).astype(jnp.float32), axis=1,
                           keepdims=True)
    n_s[...] += jnp.sum(same.astype(jnp.float32), axis=1, keepdims=True)

  @pl.when(j == NJ - 1)
  def _():
    k = jnp.ceil(jnp.float32(RATIO) * n_s[...])
    keep = (vrow[...] > 0.0) & (rank_s[...] < k)
    keep_out[...] = keep.astype(jnp.float32)


def _rank(score, batch, valid, NP, BR=1024, BJ=2048):
  NI, NJ = NP // BR, NP // BJ
  scol = score.reshape(1, NP)
  bcol = batch.reshape(1, NP)
  vcol = valid.reshape(1, NP)
  return pl.pallas_call(
      functools.partial(_rank_block, BR=BR, BJ=BJ, NJ=NJ),
      grid=(NI, NJ),
      in_specs=[
          pl.BlockSpec((BR, 1), lambda i, j: (i, 0)),
          pl.BlockSpec((BR, 1), lambda i, j: (i, 0)),
          pl.BlockSpec((BR, 1), lambda i, j: (i, 0)),
          pl.BlockSpec((1, BJ), lambda i, j: (0, j)),
          pl.BlockSpec((1, BJ), lambda i, j: (0, j)),
          pl.BlockSpec((1, BJ), lambda i, j: (0, j)),
      ],
      out_specs=pl.BlockSpec((BR, 1), lambda i, j: (i, 0)),
      out_shape=jax.ShapeDtypeStruct((NP, 1), jnp.float32),
      scratch_shapes=[
          pltpu.VMEM((BR, 1), jnp.float32),
          pltpu.VMEM((BR, 1), jnp.float32),
      ],
  )(score, batch.reshape(NP, 1), valid, scol, bcol, vcol)


def _readout_block(h, keep, score, batch, xn_out, ro_out, mx_s, sm_s, cnt_s,
                   *, NB, F):
  i = pl.program_id(0)

  @pl.when(i == 0)
  def _():
    mx_s[...] = jnp.full_like(mx_s, -jnp.inf)
    sm_s[...] = jnp.zeros_like(sm_s)
    cnt_s[...] = jnp.zeros_like(cnt_s)

  kp = keep[...] > 0.0
  xn = jnp.where(kp, h[...] * score[...], 0.0)
  xn_out[...] = xn
  b = batch[...]
  gids = lax.broadcasted_iota(jnp.int32, (1, NG), 1)
  onehot = ((b == gids) & kp).astype(jnp.float32)
  sm_s[...] += lax.dot_general(onehot, xn, (((0,), (0,)), ((), ())),
                               preferred_element_type=jnp.float32)
  cnt_s[...] += lax.dot_general(onehot, jnp.ones_like(keep[...]),
                                (((0,), (0,)), ((), ())),
                                preferred_element_type=jnp.float32)
  def gmax(g, carry):
    mask = (b == g) & kp
    mval = jnp.max(jnp.where(mask, xn, -jnp.inf), axis=0, keepdims=True)
    cur = mx_s[pl.ds(g, 1), :]
    mx_s[pl.ds(g, 1), :] = jnp.maximum(cur, mval)
    return carry

  lax.fori_loop(jnp.min(b), jnp.max(b) + 1, gmax, 0)

  @pl.when(i == NB - 1)
  def _():
    mean = sm_s[...] / jnp.maximum(cnt_s[...], 1.0)
    ro_out[:, :F] = mx_s[...]
    ro_out[:, F:] = mean


def _readout(h, keep, score, batch, NP, F, BD=512):
  NB = NP // BD
  return pl.pallas_call(
      functools.partial(_readout_block, NB=NB, F=F),
      grid=(NB,),
      in_specs=[
          pl.BlockSpec((BD, F), lambda i: (i, 0)),
          pl.BlockSpec((BD, 1), lambda i: (i, 0)),
          pl.BlockSpec((BD, 1), lambda i: (i, 0)),
          pl.BlockSpec((BD, 1), lambda i: (i, 0)),
      ],
      out_specs=[
          pl.BlockSpec((BD, F), lambda i: (i, 0)),
          pl.BlockSpec((NG, 2 * F), lambda i: (0, 0)),
      ],
      out_shape=[
          jax.ShapeDtypeStruct((NP, F), jnp.float32),
          jax.ShapeDtypeStruct((NG, 2 * F), jnp.float32),
      ],
      scratch_shapes=[
          pltpu.VMEM((NG, F), jnp.float32),
          pltpu.VMEM((NG, F), jnp.float32),
          pltpu.VMEM((NG, 1), jnp.float32),
      ],
  )(h, keep, score, batch)


def _head_block(x1, x2, x3, w1t, b1, w2t, b2, w3t, b3, out):
  z = x1[...] + x2[...] + x3[...]
  z = jnp.maximum(jnp.dot(z, w1t[...], preferred_element_type=jnp.float32)
                  + b1[...], 0.0)
  z = jnp.maximum(jnp.dot(z, w2t[...], preferred_element_type=jnp.float32)
                  + b2[...], 0.0)
  z = jnp.maximum(jnp.dot(z, w3t[...], preferred_element_type=jnp.float32)
                  + b3[...], 0.0)
  out[...] = z


def _head(x1, x2, x3, W1, lb1, W2, lb2, W3, lb3):
  OUT = W3.shape[0]
  return pl.pallas_call(
      _head_block,
      out_shape=jax.ShapeDtypeStruct((NG, OUT), jnp.float32),
  )(x1, x2, x3, W1.T, lb1.reshape(1, -1), W2.T, lb2.reshape(1, -1),
    W3.T, lb3.reshape(1, -1))


def kernel(x, edge_index, batch, Wl1, bl1, Wr1, w1, Wl2, bl2, Wr2, w2,
           Wl3, bl3, Wr3, w3, W1, lb1, W2, lb2, W3, lb3):
  N = x.shape[0]
  F = x.shape[-1]
  E = edge_index.shape[1]

  NP = ((N + 1 + 511) // 512) * 512
  CPT = (E + NW * CHUNK - 1) // (NW * CHUNK)
  CPT = CPT + (CPT % 2)
  EP = NW * CPT * CHUNK

  xp = jnp.zeros((NP, F), jnp.float32).at[:N].set(x.reshape(N, F))
  batch_p = jnp.full((NP,), NG - 1, jnp.int32).at[:N].set(
      batch.astype(jnp.int32))
  valid = (jnp.arange(NP) < N).astype(jnp.float32)
  padidx = (N + (jnp.arange(EP - E) % min(CHUNK, NP - N - 1))).astype(jnp.int32)
  src = jnp.concatenate([edge_index[0].astype(jnp.int32), padidx])
  dst = jnp.concatenate([edge_index[1].astype(jnp.int32), padidx])
  srcg = src.reshape(NW, CPT, CHUNK)
  dstg = dst.reshape(NW, CPT, CHUNK)
  zf = jnp.zeros((NP, F), jnp.float32)
  z1 = jnp.zeros((NP,), jnp.float32)

  aggregate = _make_aggregate(NP, F, CPT)

  def layer(xc, fc, Wl, bl, Wr, w):
    aggp, degp = aggregate(xc, fc, srcg, dstg, zf, z1)
    wsc = (w / jnp.linalg.norm(w)).reshape(F, 1)
    h, score = _linear(aggp, degp.reshape(NC, NP, 1), xc, Wl.T,
                       bl.reshape(1, F), Wr.T, wsc, NP, F)
    keep = _rank(score, batch_p, fc.reshape(NP, 1), NP)
    xn, ro = _readout(h, keep, score, batch_p.reshape(NP, 1), NP, F)
    return xn, keep.reshape(NP), ro

  h1, v1, x1 = layer(xp, valid, Wl1, bl1, Wr1, w1)
  h2, v2, x2 = layer(h1, v1, Wl2, bl2, Wr2, w2)
  _, _, x3 = layer(h2, v2, Wl3, bl3, Wr3, w3)
  return _head(x1, x2, x3, W1, lb1, W2, lb2, W3, lb3)


# speedup vs baseline: 15.9834x; 1.0250x over previous
import functools

import jax
import jax.numpy as jnp
from jax import lax
from jax.experimental import pallas as pl
from jax.experimental.pallas import tpu as pltpu
from jax.experimental.pallas import tpu_sc as plsc

NG = 64
RATIO = 0.8
NC = 2
NS = 16
NW = NC * NS
CHUNK = 128


def _make_aggregate(NP, F, CPT):
  RPT = NP // NS

  mesh = plsc.VectorSubcoreMesh(core_axis_name="c", subcore_axis_name="s")

  @functools.partial(
      pl.kernel,
      out_type=(
          jax.ShapeDtypeStruct((NC, NP, F), jnp.float32),
          jax.ShapeDtypeStruct((NC, NP), jnp.float32),
      ),
      mesh=mesh,
      scratch_types=[
          pltpu.VMEM((2, CHUNK), jnp.int32),
          pltpu.VMEM((2, CHUNK), jnp.int32),
          pltpu.VMEM((2, CHUNK, F), jnp.float32),
          pltpu.VMEM((2, CHUNK), jnp.float32),
          pltpu.VMEM_SHARED((NP, F), jnp.float32),
          pltpu.VMEM_SHARED((NP,), jnp.float32),
          pltpu.SemaphoreType.DMA((2,)),
          pltpu.SemaphoreType.DMA((2,)),
          pltpu.SemaphoreType.DMA((2,)),
      ],
  )
  def agg_kernel(x_hbm, f_hbm, srcg, dstg, zf_hbm, z1_hbm,
                 aggp, degp,
                 src_v, dst_v, rows_v, fv, agg_sh, deg_sh, sem, sem2, sem3):
    c = lax.axis_index("c")
    s = lax.axis_index("s")
    wid = c * NS + s

    pltpu.sync_copy(zf_hbm.at[pl.ds(s * RPT, RPT), :],
                    agg_sh.at[pl.ds(s * RPT, RPT), :])
    pltpu.sync_copy(z1_hbm.at[pl.ds(s * RPT, RPT)],
                    deg_sh.at[pl.ds(s * RPT, RPT)])
    plsc.subcore_barrier()

    def idx_issue(i, b):
      pltpu.async_copy(srcg.at[wid, i], src_v.at[b], sem3.at[b])
      pltpu.async_copy(dstg.at[wid, i], dst_v.at[b], sem3.at[b])

    def idx_wait(i, b):
      pltpu.make_async_copy(srcg.at[wid, i], src_v.at[b], sem3.at[b]).wait()
      pltpu.make_async_copy(dstg.at[wid, i], dst_v.at[b], sem3.at[b]).wait()

    def g_issue(i, b):
      pltpu.async_copy(x_hbm.at[src_v.at[b]], rows_v.at[b], sem.at[b])
      pltpu.async_copy(f_hbm.at[src_v.at[b]], fv.at[b], sem2.at[b])

    def g_wait(i, b):
      pltpu.make_async_copy(x_hbm.at[src_v.at[b]], rows_v.at[b],
                            sem.at[b]).wait()
      pltpu.make_async_copy(f_hbm.at[src_v.at[b]], fv.at[b],
                            sem2.at[b]).wait()

    idx_issue(0, 0)
    idx_wait(0, 0)
    g_issue(0, 0)

    @pl.loop(0, CPT // 2)
    def _(ih):
      for b in range(2):
        i = ih * 2 + b

        @pl.when(i < CPT - 1)
        def _():
          idx_issue(i + 1, 1 - b)

        g_wait(i, b)

        @pl.when(i < CPT - 1)
        def _():
          idx_wait(i + 1, 1 - b)
          g_issue(i + 1, 1 - b)

        pltpu.sync_copy(rows_v.at[b], agg_sh.at[dst_v.at[b]], add=True)
        pltpu.sync_copy(fv.at[b], deg_sh.at[dst_v.at[b]], add=True)

    plsc.subcore_barrier()
    pltpu.sync_copy(agg_sh.at[pl.ds(s * RPT, RPT), :],
                    aggp.at[c, pl.ds(s * RPT, RPT), :])
    pltpu.sync_copy(deg_sh.at[pl.ds(s * RPT, RPT)],
                    degp.at[c, pl.ds(s * RPT, RPT)])

  return agg_kernel


def _linear_block(agg0, agg1, deg0, deg1, x, wlt, bl, wrt, wsc, h_out, s_out):
  deg = jnp.maximum(deg0[...] + deg1[...], 1.0)
  agg = (agg0[...] + agg1[...]) / deg
  h = jnp.dot(agg, wlt[...], preferred_element_type=jnp.float32)
  h = h + bl[...] + jnp.dot(x[...], wrt[...], preferred_element_type=jnp.float32)
  h = jnp.maximum(h, 0.0)
  h_out[...] = h
  s_out[...] = jnp.tanh(jnp.dot(h, wsc[...], preferred_element_type=jnp.float32))


def _linear(aggp, degp, x, wlt, bl, wrt, wsc, NP, F, BN=1024):
  grid = (NP // BN,)
  return pl.pallas_call(
      _linear_block,
      grid=grid,
      in_specs=[
          pl.BlockSpec((None, BN, F), lambda i: (0, i, 0)),
          pl.BlockSpec((None, BN, F), lambda i: (1, i, 0)),
          pl.BlockSpec((None, BN, 1), lambda i: (0, i, 0)),
          pl.BlockSpec((None, BN, 1), lambda i: (1, i, 0)),
          pl.BlockSpec((BN, F), lambda i: (i, 0)),
          pl.BlockSpec((F, F), lambda i: (0, 0)),
          pl.BlockSpec((1, F), lambda i: (0, 0)),
          pl.BlockSpec((F, F), lambda i: (0, 0)),
          pl.BlockSpec((F, 1), lambda i: (0, 0)),
      ],
      out_specs=[
          pl.BlockSpec((BN, F), lambda i: (i, 0)),
          pl.BlockSpec((BN, 1), lambda i: (i, 0)),
      ],
      out_shape=[
          jax.ShapeDtypeStruct((NP, F), jnp.float32),
          jax.ShapeDtypeStruct((NP, 1), jnp.float32),
      ],
  )(aggp, aggp, degp, degp, x, wlt, bl, wrt, wsc)


def _rank_block(srow, brow, vrow, scol, bcol, vcol, keep_out, rank_s, n_s,
                *, BR, BJ, NJ):
  i = pl.program_id(0)
  j = pl.program_id(1)

  @pl.when(j == 0)
  def _():
    rank_s[...] = jnp.zeros_like(rank_s)
    n_s[...] = jnp.zeros_like(n_s)

  bi = brow[...]
  bj = bcol[...]

  @pl.when((jnp.min(bj) <= jnp.max(bi)) & (jnp.max(bj) >= jnp.min(bi)))
  def _():
    si = srow[...]
    sj = scol[...]
    vj = vcol[...] > 0.0
    ridx = i * BR + lax.broadcasted_iota(jnp.int32, (BR, 1), 0)
    cidx = j * BJ + lax.broadcasted_iota(jnp.int32, (1, BJ), 1)
    same = (bj == bi) & vj
    ahead = (sj > si) | ((sj == si) & (cidx < ridx))
    rank_s[...] += jnp.sum((same & ahead).astype(jnp.float32), axis=1,
                           keepdims=True)
    n_s[...] += jnp.sum(same.astype(jnp.float32), axis=1, keepdims=True)

  @pl.when(j == NJ - 1)
  def _():
    k = jnp.ceil(jnp.float32(RATIO) * n_s[...])
    keep = (vrow[...] > 0.0) & (rank_s[...] < k)
    keep_out[...] = keep.astype(jnp.float32)


def _rank(score, batch, valid, NP, BR=1024, BJ=2048):
  NI, NJ = NP // BR, NP // BJ
  scol = score.reshape(1, NP)
  bcol = batch.reshape(1, NP)
  vcol = valid.reshape(1, NP)
  return pl.pallas_call(
      functools.partial(_rank_block, BR=BR, BJ=BJ, NJ=NJ),
      grid=(NI, NJ),
      in_specs=[
          pl.BlockSpec((BR, 1), lambda i, j: (i, 0)),
          pl.BlockSpec((BR, 1), lambda i, j: (i, 0)),
          pl.BlockSpec((BR, 1), lambda i, j: (i, 0)),
          pl.BlockSpec((1, BJ), lambda i, j: (0, j)),
          pl.BlockSpec((1, BJ), lambda i, j: (0, j)),
          pl.BlockSpec((1, BJ), lambda i, j: (0, j)),
      ],
      out_specs=pl.BlockSpec((BR, 1), lambda i, j: (i, 0)),
      out_shape=jax.ShapeDtypeStruct((NP, 1), jnp.float32),
      scratch_shapes=[
          pltpu.VMEM((BR, 1), jnp.float32),
          pltpu.VMEM((BR, 1), jnp.float32),
      ],
  )(score, batch.reshape(NP, 1), valid, scol, bcol, vcol)


def _readout_block(h, keep, score, batch, xn_out, ro_out, mx_s, sm_s, cnt_s,
                   *, NB, F):
  i = pl.program_id(0)

  @pl.when(i == 0)
  def _():
    mx_s[...] = jnp.full_like(mx_s, -jnp.inf)
    sm_s[...] = jnp.zeros_like(sm_s)
    cnt_s[...] = jnp.zeros_like(cnt_s)

  kp = keep[...] > 0.0
  xn = jnp.where(kp, h[...] * score[...], 0.0)
  xn_out[...] = xn
  b = batch[...]
  gids = lax.broadcasted_iota(jnp.int32, (1, NG), 1)
  onehot = ((b == gids) & kp).astype(jnp.float32)
  sm_s[...] += lax.dot_general(onehot, xn, (((0,), (0,)), ((), ())),
                               preferred_element_type=jnp.float32)
  cnt_s[...] += lax.dot_general(onehot, jnp.ones_like(keep[...]),
                                (((0,), (0,)), ((), ())),
                                preferred_element_type=jnp.float32)
  def gmax(g, carry):
    mask = (b == g) & kp
    mval = jnp.max(jnp.where(mask, xn, -jnp.inf), axis=0, keepdims=True)
    cur = mx_s[pl.ds(g, 1), :]
    mx_s[pl.ds(g, 1), :] = jnp.maximum(cur, mval)
    return carry

  lax.fori_loop(jnp.min(b), jnp.max(b) + 1, gmax, 0)

  @pl.when(i == NB - 1)
  def _():
    mean = sm_s[...] / jnp.maximum(cnt_s[...], 1.0)
    ro_out[:, :F] = mx_s[...]
    ro_out[:, F:] = mean


def _readout(h, keep, score, batch, NP, F, BD=1024):
  NB = NP // BD
  return pl.pallas_call(
      functools.partial(_readout_block, NB=NB, F=F),
      grid=(NB,),
      in_specs=[
          pl.BlockSpec((BD, F), lambda i: (i, 0)),
          pl.BlockSpec((BD, 1), lambda i: (i, 0)),
          pl.BlockSpec((BD, 1), lambda i: (i, 0)),
          pl.BlockSpec((BD, 1), lambda i: (i, 0)),
      ],
      out_specs=[
          pl.BlockSpec((BD, F), lambda i: (i, 0)),
          pl.BlockSpec((NG, 2 * F), lambda i: (0, 0)),
      ],
      out_shape=[
          jax.ShapeDtypeStruct((NP, F), jnp.float32),
          jax.ShapeDtypeStruct((NG, 2 * F), jnp.float32),
      ],
      scratch_shapes=[
          pltpu.VMEM((NG, F), jnp.float32),
          pltpu.VMEM((NG, F), jnp.float32),
          pltpu.VMEM((NG, 1), jnp.float32),
      ],
  )(h, keep, score, batch)


def _head_block(x1, x2, x3, w1t, b1, w2t, b2, w3t, b3, out):
  z = x1[...] + x2[...] + x3[...]
  z = jnp.maximum(jnp.dot(z, w1t[...], preferred_element_type=jnp.float32)
                  + b1[...], 0.0)
  z = jnp.maximum(jnp.dot(z, w2t[...], preferred_element_type=jnp.float32)
                  + b2[...], 0.0)
  z = jnp.maximum(jnp.dot(z, w3t[...], preferred_element_type=jnp.float32)
                  + b3[...], 0.0)
  out[...] = z


def _head(x1, x2, x3, W1, lb1, W2, lb2, W3, lb3):
  OUT = W3.shape[0]
  return pl.pallas_call(
      _head_block,
      out_shape=jax.ShapeDtypeStruct((NG, OUT), jnp.float32),
  )(x1, x2, x3, W1.T, lb1.reshape(1, -1), W2.T, lb2.reshape(1, -1),
    W3.T, lb3.reshape(1, -1))


def kernel(x, edge_index, batch, Wl1, bl1, Wr1, w1, Wl2, bl2, Wr2, w2,
           Wl3, bl3, Wr3, w3, W1, lb1, W2, lb2, W3, lb3):
  N = x.shape[0]
  F = x.shape[-1]
  E = edge_index.shape[1]

  NP = ((N + 1 + 511) // 512) * 512
  CPT = (E + NW * CHUNK - 1) // (NW * CHUNK)
  CPT = CPT + (CPT % 2)
  EP = NW * CPT * CHUNK

  xp = jnp.zeros((NP, F), jnp.float32).at[:N].set(x.reshape(N, F))
  batch_p = jnp.full((NP,), NG - 1, jnp.int32).at[:N].set(
      batch.astype(jnp.int32))
  valid = (jnp.arange(NP) < N).astype(jnp.float32)
  padidx = (N + (jnp.arange(EP - E) % min(CHUNK, NP - N - 1))).astype(jnp.int32)
  src = jnp.concatenate([edge_index[0].astype(jnp.int32), padidx])
  dst = jnp.concatenate([edge_index[1].astype(jnp.int32), padidx])
  srcg = src.reshape(NW, CPT, CHUNK)
  dstg = dst.reshape(NW, CPT, CHUNK)
  zf = jnp.zeros((NP, F), jnp.float32)
  z1 = jnp.zeros((NP,), jnp.float32)

  aggregate = _make_aggregate(NP, F, CPT)

  def layer(xc, fc, Wl, bl, Wr, w):
    aggp, degp = aggregate(xc, fc, srcg, dstg, zf, z1)
    wsc = (w / jnp.linalg.norm(w)).reshape(F, 1)
    h, score = _linear(aggp, degp.reshape(NC, NP, 1), xc, Wl.T,
                       bl.reshape(1, F), Wr.T, wsc, NP, F)
    keep = _rank(score, batch_p, fc.reshape(NP, 1), NP)
    xn, ro = _readout(h, keep, score, batch_p.reshape(NP, 1), NP, F)
    return xn, keep.reshape(NP), ro

  h1, v1, x1 = layer(xp, valid, Wl1, bl1, Wr1, w1)
  h2, v2, x2 = layer(h1, v1, Wl2, bl2, Wr2, w2)
  _, _, x3 = layer(h2, v2, Wl3, bl3, Wr3, w3)
  return _head(x1, x2, x3, W1, lb1, W2, lb2, W3, lb3)


# speedup vs baseline: 16.2666x; 1.0177x over previous
import functools

import jax
import jax.numpy as jnp
from jax import lax
from jax.experimental import pallas as pl
from jax.experimental.pallas import tpu as pltpu
from jax.experimental.pallas import tpu_sc as plsc

NG = 64
RATIO = 0.8
NC = 2
NS = 16
NW = NC * NS
CHUNK = 128


def _make_aggregate(NP, F, CPT):
  RPT = NP // NS

  mesh = plsc.VectorSubcoreMesh(core_axis_name="c", subcore_axis_name="s")

  @functools.partial(
      pl.kernel,
      out_type=(
          jax.ShapeDtypeStruct((NC, NP, F), jnp.float32),
          jax.ShapeDtypeStruct((NC, NP), jnp.float32),
      ),
      mesh=mesh,
      scratch_types=[
          pltpu.VMEM((2, CHUNK), jnp.int32),
          pltpu.VMEM((2, CHUNK), jnp.int32),
          pltpu.VMEM((2, CHUNK, F), jnp.float32),
          pltpu.VMEM((2, CHUNK), jnp.float32),
          pltpu.VMEM_SHARED((NP, F), jnp.float32),
          pltpu.VMEM_SHARED((NP,), jnp.float32),
          pltpu.SemaphoreType.DMA((2,)),
          pltpu.SemaphoreType.DMA((2,)),
          pltpu.SemaphoreType.DMA((2,)),
      ],
  )
  def agg_kernel(x_hbm, f_hbm, srcg, dstg, zf_hbm, z1_hbm,
                 aggp, degp,
                 src_v, dst_v, rows_v, fv, agg_sh, deg_sh, sem, sem2, sem3):
    c = lax.axis_index("c")
    s = lax.axis_index("s")
    wid = c * NS + s

    pltpu.sync_copy(zf_hbm.at[pl.ds(s * RPT, RPT), :],
                    agg_sh.at[pl.ds(s * RPT, RPT), :])
    pltpu.sync_copy(z1_hbm.at[pl.ds(s * RPT, RPT)],
                    deg_sh.at[pl.ds(s * RPT, RPT)])
    plsc.subcore_barrier()

    def idx_issue(i, b):
      pltpu.async_copy(srcg.at[wid, i], src_v.at[b], sem3.at[b])
      pltpu.async_copy(dstg.at[wid, i], dst_v.at[b], sem3.at[b])

    def idx_wait(i, b):
      pltpu.make_async_copy(srcg.at[wid, i], src_v.at[b], sem3.at[b]).wait()
      pltpu.make_async_copy(dstg.at[wid, i], dst_v.at[b], sem3.at[b]).wait()

    def g_issue(i, b):
      pltpu.async_copy(x_hbm.at[src_v.at[b]], rows_v.at[b], sem.at[b])
      pltpu.async_copy(f_hbm.at[src_v.at[b]], fv.at[b], sem2.at[b])

    def g_wait(i, b):
      pltpu.make_async_copy(x_hbm.at[src_v.at[b]], rows_v.at[b],
                            sem.at[b]).wait()
      pltpu.make_async_copy(f_hbm.at[src_v.at[b]], fv.at[b],
                            sem2.at[b]).wait()

    idx_issue(0, 0)
    idx_wait(0, 0)
    g_issue(0, 0)

    @pl.loop(0, CPT // 2)
    def _(ih):
      for b in range(2):
        i = ih * 2 + b

        @pl.when(i < CPT - 1)
        def _():
          idx_issue(i + 1, 1 - b)

        g_wait(i, b)

        @pl.when(i < CPT - 1)
        def _():
          idx_wait(i + 1, 1 - b)
          g_issue(i + 1, 1 - b)

        pltpu.sync_copy(rows_v.at[b], agg_sh.at[dst_v.at[b]], add=True)
        pltpu.sync_copy(fv.at[b], deg_sh.at[dst_v.at[b]], add=True)

    plsc.subcore_barrier()
    pltpu.sync_copy(agg_sh.at[pl.ds(s * RPT, RPT), :],
                    aggp.at[c, pl.ds(s * RPT, RPT), :])
    pltpu.sync_copy(deg_sh.at[pl.ds(s * RPT, RPT)],
                    degp.at[c, pl.ds(s * RPT, RPT)])

  return agg_kernel


def _linear_block(agg0, agg1, deg0, deg1, x, wlt, bl, wrt, wsc, h_out, s_out):
  deg = jnp.maximum(deg0[...] + deg1[...], 1.0)
  agg = (agg0[...] + agg1[...]) / deg
  h = jnp.dot(agg, wlt[...], preferred_element_type=jnp.float32)
  h = h + bl[...] + jnp.dot(x[...], wrt[...], preferred_element_type=jnp.float32)
  h = jnp.maximum(h, 0.0)
  h_out[...] = h
  s_out[...] = jnp.tanh(jnp.dot(h, wsc[...], preferred_element_type=jnp.float32))


def _linear(aggp, degp, x, wlt, bl, wrt, wsc, NP, F, BN=1024):
  grid = (NP // BN,)
  return pl.pallas_call(
      _linear_block,
      grid=grid,
      in_specs=[
          pl.BlockSpec((None, BN, F), lambda i: (0, i, 0)),
          pl.BlockSpec((None, BN, F), lambda i: (1, i, 0)),
          pl.BlockSpec((None, BN, 1), lambda i: (0, i, 0)),
          pl.BlockSpec((None, BN, 1), lambda i: (1, i, 0)),
          pl.BlockSpec((BN, F), lambda i: (i, 0)),
          pl.BlockSpec((F, F), lambda i: (0, 0)),
          pl.BlockSpec((1, F), lambda i: (0, 0)),
          pl.BlockSpec((F, F), lambda i: (0, 0)),
          pl.BlockSpec((F, 1), lambda i: (0, 0)),
      ],
      out_specs=[
          pl.BlockSpec((BN, F), lambda i: (i, 0)),
          pl.BlockSpec((BN, 1), lambda i: (i, 0)),
      ],
      out_shape=[
          jax.ShapeDtypeStruct((NP, F), jnp.float32),
          jax.ShapeDtypeStruct((NP, 1), jnp.float32),
      ],
  )(aggp, aggp, degp, degp, x, wlt, bl, wrt, wsc)


def _rankro_block(srow, brow, vrow, scol, bcol, vcol, h,
                  keep_out, xn_out, ro_out,
                  rank_s, n_s, mx_s, sm_s, cnt_s, *, BR, BJ, NI, NJ, F):
  i = pl.program_id(0)
  j = pl.program_id(1)

  @pl.when(j == 0)
  def _():
    rank_s[...] = jnp.zeros_like(rank_s)
    n_s[...] = jnp.zeros_like(n_s)

  @pl.when((i == 0) & (j == 0))
  def _():
    mx_s[...] = jnp.full_like(mx_s, -jnp.inf)
    sm_s[...] = jnp.zeros_like(sm_s)
    cnt_s[...] = jnp.zeros_like(cnt_s)

  bi = brow[...]
  bj = bcol[...]

  @pl.when((jnp.min(bj) <= jnp.max(bi)) & (jnp.max(bj) >= jnp.min(bi)))
  def _():
    si = srow[...]
    sj = scol[...]
    vj = vcol[...] > 0.0
    ridx = i * BR + lax.broadcasted_iota(jnp.int32, (BR, 1), 0)
    cidx = j * BJ + lax.broadcasted_iota(jnp.int32, (1, BJ), 1)
    same = (bj == bi) & vj
    ahead = (sj > si) | ((sj == si) & (cidx < ridx))
    rank_s[...] += jnp.sum((same & ahead).astype(jnp.float32), axis=1,
                           keepdims=True)
    n_s[...] += jnp.sum(same.astype(jnp.float32), axis=1, keepdims=True)

  @pl.when(j == NJ - 1)
  def _():
    k = jnp.ceil(jnp.float32(RATIO) * n_s[...])
    kp = (vrow[...] > 0.0) & (rank_s[...] < k)
    keep_out[...] = kp.astype(jnp.float32)
    xn = jnp.where(kp, h[...] * srow[...], 0.0)
    xn_out[...] = xn
    gids = lax.broadcasted_iota(jnp.int32, (1, NG), 1)
    onehot = ((bi == gids) & kp).astype(jnp.float32)
    sm_s[...] += lax.dot_general(onehot, xn, (((0,), (0,)), ((), ())),
                                 preferred_element_type=jnp.float32)
    cnt_s[...] += lax.dot_general(onehot, kp.astype(jnp.float32),
                                  (((0,), (0,)), ((), ())),
                                  preferred_element_type=jnp.float32)

    def gmax(g, carry):
      mask = (bi == g) & kp
      mval = jnp.max(jnp.where(mask, xn, -jnp.inf), axis=0, keepdims=True)
      cur = mx_s[pl.ds(g, 1), :]
      mx_s[pl.ds(g, 1), :] = jnp.maximum(cur, mval)
      return carry

    lax.fori_loop(jnp.min(bi), jnp.max(bi) + 1, gmax, 0)

  @pl.when((i == NI - 1) & (j == NJ - 1))
  def _():
    mean = sm_s[...] / jnp.maximum(cnt_s[...], 1.0)
    ro_out[:, :F] = mx_s[...]
    ro_out[:, F:] = mean


def _rankro(score, batch, valid, h, NP, F, BR=1024, BJ=2048):
  NI, NJ = NP // BR, NP // BJ
  scol = score.reshape(1, NP)
  bcol = batch.reshape(1, NP)
  vcol = valid.reshape(1, NP)
  return pl.pallas_call(
      functools.partial(_rankro_block, BR=BR, BJ=BJ, NI=NI, NJ=NJ, F=F),
      grid=(NI, NJ),
      in_specs=[
          pl.BlockSpec((BR, 1), lambda i, j: (i, 0)),
          pl.BlockSpec((BR, 1), lambda i, j: (i, 0)),
          pl.BlockSpec((BR, 1), lambda i, j: (i, 0)),
          pl.BlockSpec((1, BJ), lambda i, j: (0, j)),
          pl.BlockSpec((1, BJ), lambda i, j: (0, j)),
          pl.BlockSpec((1, BJ), lambda i, j: (0, j)),
          pl.BlockSpec((BR, F), lambda i, j: (i, 0)),
      ],
      out_specs=[
          pl.BlockSpec((BR, 1), lambda i, j: (i, 0)),
          pl.BlockSpec((BR, F), lambda i, j: (i, 0)),
          pl.BlockSpec((NG, 2 * F), lambda i, j: (0, 0)),
      ],
      out_shape=[
          jax.ShapeDtypeStruct((NP, 1), jnp.float32),
          jax.ShapeDtypeStruct((NP, F), jnp.float32),
          jax.ShapeDtypeStruct((NG, 2 * F), jnp.float32),
      ],
      scratch_shapes=[
          pltpu.VMEM((BR, 1), jnp.float32),
          pltpu.VMEM((BR, 1), jnp.float32),
          pltpu.VMEM((NG, F), jnp.float32),
          pltpu.VMEM((NG, F), jnp.float32),
          pltpu.VMEM((NG, 1), jnp.float32),
      ],
  )(score, batch.reshape(NP, 1), valid, scol, bcol, vcol, h)


def _head_block(x1, x2, x3, w1t, b1, w2t, b2, w3t, b3, out):
  z = x1[...] + x2[...] + x3[...]
  z = jnp.maximum(jnp.dot(z, w1t[...], preferred_element_type=jnp.float32)
                  + b1[...], 0.0)
  z = jnp.maximum(jnp.dot(z, w2t[...], preferred_element_type=jnp.float32)
                  + b2[...], 0.0)
  z = jnp.maximum(jnp.dot(z, w3t[...], preferred_element_type=jnp.float32)
                  + b3[...], 0.0)
  out[...] = z


def _head(x1, x2, x3, W1, lb1, W2, lb2, W3, lb3):
  OUT = W3.shape[0]
  return pl.pallas_call(
      _head_block,
      out_shape=jax.ShapeDtypeStruct((NG, OUT), jnp.float32),
  )(x1, x2, x3, W1.T, lb1.reshape(1, -1), W2.T, lb2.reshape(1, -1),
    W3.T, lb3.reshape(1, -1))


def kernel(x, edge_index, batch, Wl1, bl1, Wr1, w1, Wl2, bl2, Wr2, w2,
           Wl3, bl3, Wr3, w3, W1, lb1, W2, lb2, W3, lb3):
  N = x.shape[0]
  F = x.shape[-1]
  E = edge_index.shape[1]

  NP = ((N + 1 + 511) // 512) * 512
  CPT = (E + NW * CHUNK - 1) // (NW * CHUNK)
  CPT = CPT + (CPT % 2)
  EP = NW * CPT * CHUNK

  xp = jnp.zeros((NP, F), jnp.float32).at[:N].set(x.reshape(N, F))
  batch_p = jnp.full((NP,), NG - 1, jnp.int32).at[:N].set(
      batch.astype(jnp.int32))
  valid = (jnp.arange(NP) < N).astype(jnp.float32)
  padidx = (N + (jnp.arange(EP - E) % min(CHUNK, NP - N - 1))).astype(jnp.int32)
  src = jnp.concatenate([edge_index[0].astype(jnp.int32), padidx])
  dst = jnp.concatenate([edge_index[1].astype(jnp.int32), padidx])
  srcg = src.reshape(NW, CPT, CHUNK)
  dstg = dst.reshape(NW, CPT, CHUNK)
  zf = jnp.zeros((NP, F), jnp.float32)
  z1 = jnp.zeros((NP,), jnp.float32)

  aggregate = _make_aggregate(NP, F, CPT)

  def layer(xc, fc, Wl, bl, Wr, w):
    aggp, degp = aggregate(xc, fc, srcg, dstg, zf, z1)
    wsc = (w / jnp.linalg.norm(w)).reshape(F, 1)
    h, score = _linear(aggp, degp.reshape(NC, NP, 1), xc, Wl.T,
                       bl.reshape(1, F), Wr.T, wsc, NP, F)
    keep, xn, ro = _rankro(score, batch_p, fc.reshape(NP, 1), h, NP, F)
    return xn, keep.reshape(NP), ro

  h1, v1, x1 = layer(xp, valid, Wl1, bl1, Wr1, w1)
  h2, v2, x2 = layer(h1, v1, Wl2, bl2, Wr2, w2)
  _, _, x3 = layer(h2, v2, Wl3, bl3, Wr3, w3)
  return _head(x1, x2, x3, W1, lb1, W2, lb2, W3, lb3)
